# baseline (speedup 1.0000x reference)
import jax
import jax.numpy as jnp
from jax.experimental import pallas as pl

K = 32


def _copy_kernel(x_ref, o_ref):
    o_ref[...] = x_ref[...]


def kernel(x, W_enc, b_enc, W_dec, b_dec):
    x = pl.pallas_call(
        _copy_kernel,
        out_shape=jax.ShapeDtypeStruct(x.shape, x.dtype),
    )(x)
    pre = jnp.einsum('btd,std->bs', x - b_dec, W_enc) + b_enc
    vals, idx = jax.lax.top_k(pre, K)
    rows = jnp.arange(pre.shape[0])[:, None]
    z = jnp.zeros_like(pre).at[rows, idx].set(jax.nn.relu(vals))
    x_hat = jnp.einsum('bs,std->btd', z, W_dec) + b_dec
    recon_loss = jnp.mean(jnp.sum((x_hat - x) ** 2, axis=-1))
    return (recon_loss, x_hat, z)


# speedup vs baseline: 9.0418x; 9.0418x over previous
import functools

import jax
import jax.numpy as jnp
from jax import lax
from jax.experimental import pallas as pl
from jax.experimental.pallas import tpu as pltpu
from jax.experimental.pallas import tpu_sc as plsc

B = 1024
T = 2
D_IN = 128
D = T * D_IN
S = 65536
K = 32
SEG = 512
NSEG = S // SEG
LEV = 10
NCAND = NSEG * LEV
NEG = float("-inf")

NW = 32
RPW = B // NW


def _encode_kernel(xmt_ref, w_ref, be_ref, mv_ref, mi_ref, work_ref):
    cb = pl.program_id(0)
    pre = lax.dot_general(
        w_ref[...], xmt_ref[...], (((1,), (0,)), ((), ())),
        preferred_element_type=jnp.float32,
        precision=lax.Precision.DEFAULT,
    )
    work_ref[...] = pre + be_ref[...].reshape(SEG, 1)
    base = cb * SEG

    def level(t, _):
        w = work_ref[...]
        m = jnp.max(w, axis=0)
        i = jnp.argmax(w, axis=0).astype(jnp.int32)
        mv_ref[0, t, 0, :] = m
        mi_ref[0, t, 0, :] = i + base
        iota0 = lax.broadcasted_iota(jnp.int32, (SEG, B), 0)
        work_ref[...] = jnp.where(iota0 == i[None, :], NEG, w)
        return 0

    lax.fori_loop(0, LEV, level, 0)


def _topk_kernel(mv_ref, mi_ref, ov_ref, oi_ref, cv_ref):
    cv_ref[...] = mv_ref[...].reshape(NCAND, B)

    def pop(p, _):
        c = cv_ref[...]
        m = jnp.max(c, axis=0)
        fiota = lax.broadcasted_iota(jnp.int32, (NCAND, B), 0)
        eq = c == m[None, :]
        fi = jnp.min(jnp.where(eq, fiota, NCAND), axis=0)
        oh = fiota == fi[None, :]
        ci = mi_ref[...].reshape(NCAND, B)
        gi = jnp.sum(jnp.where(oh, ci, 0), axis=0)
        ov_ref[p, :] = m
        oi_ref[p, :] = gi
        cv_ref[...] = jnp.where(oh, NEG, c)
        return 0

    lax.fori_loop(0, K, pop, 0)


def _sc_zdec_kernel(vals_hbm, idx_hbm, wd_hbm, x_hbm, bd_hbm,
                    z_hbm, xh_hbm, lp_hbm,
                    zbuf, gbuf, vbuf, ibuf, irow, xbuf, bdbuf, acc, lacc,
                    gsem, zsem):
    wid = lax.axis_index("s") * 2 + lax.axis_index("c")
    r0 = wid * RPW
    pltpu.sync_copy(vals_hbm.at[pl.ds(r0 * K, RPW * K)], vbuf.at[pl.ds(0, RPW * K)])
    pltpu.sync_copy(idx_hbm.at[pl.ds(r0, RPW)], ibuf)
    pltpu.sync_copy(x_hbm.at[pl.ds(r0, RPW)], xbuf)
    pltpu.sync_copy(bd_hbm, bdbuf)
    lacc[...] = jnp.zeros((16,), jnp.float32)

    def zero16(i, _):
        zbuf[pl.ds(i * 16, 16)] = jnp.zeros((16,), jnp.float32)
        return 0

    lax.fori_loop(0, S // 16, zero16, 0)

    def row(r, _):
        irow[pl.ds(0, 16)] = ibuf[r, pl.ds(0, 16)]
        irow[pl.ds(16, 16)] = ibuf[r, pl.ds(16, 16)]
        gcopy = pltpu.async_copy(wd_hbm.at[irow], gbuf, gsem)
        iv0 = irow[pl.ds(0, 16)]
        iv1 = irow[pl.ds(16, 16)]
        rv0 = jnp.maximum(vbuf[pl.ds(r * K, 16)], 0.0)
        rv1 = jnp.maximum(vbuf[pl.ds(r * K + 16, 16)], 0.0)
        plsc.store_scatter(zbuf, [iv0], rv0)
        plsc.store_scatter(zbuf, [iv1], rv1)
        zcopy = pltpu.async_copy(zbuf, z_hbm.at[r0 + r], zsem)

        gcopy.wait()
        for c in range(D // 16):
            acc[pl.ds(c * 16, 16)] = bdbuf[pl.ds(c * 16, 16)]

        def dec(k, _):
            rvk = jnp.maximum(vbuf[pl.ds(r * K + k, 16)][0], 0.0)
            sp = jnp.full((16,), rvk, jnp.float32)
            for c in range(D // 16):
                sl = pl.ds(c * 16, 16)
                acc[sl] = acc[sl] + sp * gbuf[k, sl]
            return 0

        lax.fori_loop(0, K, dec, 0)
        for c in range(D // 16):
            sl = pl.ds(c * 16, 16)
            d = acc[sl] - xbuf[r, sl]
            lacc[...] = lacc[...] + d * d
        pltpu.sync_copy(acc, xh_hbm.at[r0 + r])

        zcopy.wait()
        z16 = jnp.zeros((16,), jnp.float32)
        plsc.store_scatter(zbuf, [iv0], z16)
        plsc.store_scatter(zbuf, [iv1], z16)
        return 0

    lax.fori_loop(0, RPW, row, 0)
    pltpu.sync_copy(lacc, lp_hbm.at[wid])


def _loss_kernel(lp_ref, loss_ref):
    loss_ref[...] = (jnp.sum(lp_ref[...]) * (1.0 / (B * T))).reshape(1, 1)


@jax.jit
def kernel(x, W_enc, b_enc, W_dec, b_dec):
    xm = (x - b_dec[None]).reshape(B, D)
    xmt = xm.T
    w2 = W_enc.reshape(S, D)
    be3 = b_enc.reshape(NSEG, SEG, 1)

    mv, mi = pl.pallas_call(
        _encode_kernel,
        grid=(NSEG,),
        in_specs=[
            pl.BlockSpec((D, B), lambda cb: (0, 0)),
            pl.BlockSpec((SEG, D), lambda cb: (cb, 0)),
            pl.BlockSpec((1, SEG, 1), lambda cb: (cb, 0, 0)),
        ],
        out_specs=[
            pl.BlockSpec((1, LEV, 1, B), lambda cb: (cb, 0, 0, 0)),
            pl.BlockSpec((1, LEV, 1, B), lambda cb: (cb, 0, 0, 0)),
        ],
        out_shape=[
            jax.ShapeDtypeStruct((NSEG, LEV, 1, B), jnp.float32),
            jax.ShapeDtypeStruct((NSEG, LEV, 1, B), jnp.int32),
        ],
        scratch_shapes=[pltpu.VMEM((SEG, B), jnp.float32)],
    )(xmt, w2, be3)

    vals_t, idx_t = pl.pallas_call(
        _topk_kernel,
        out_shape=[
            jax.ShapeDtypeStruct((K, B), jnp.float32),
            jax.ShapeDtypeStruct((K, B), jnp.int32),
        ],
        scratch_shapes=[pltpu.VMEM((NCAND, B), jnp.float32)],
    )(mv, mi)
    vals = vals_t.T
    idx = idx_t.T

    z, xh, lp = _sc_zdec(vals, idx, W_dec.reshape(S, D), x.reshape(B, D),
                         b_dec.reshape(D))

    loss = pl.pallas_call(
        _loss_kernel,
        out_shape=jax.ShapeDtypeStruct((1, 1), jnp.float32),
    )(lp)

    return (loss.reshape(()), xh.reshape(B, T, D_IN), z)


def _sc_zdec(vals, idx, wd2, x2, bd):
    mesh = plsc.VectorSubcoreMesh(core_axis_name="c", subcore_axis_name="s")
    f = pl.kernel(
        _sc_zdec_kernel, mesh=mesh,
        compiler_params=pltpu.CompilerParams(needs_layout_passes=False),
        out_type=[
            jax.ShapeDtypeStruct((B, S), jnp.float32),
            jax.ShapeDtypeStruct((B, D), jnp.float32),
            jax.ShapeDtypeStruct((NW, 16), jnp.float32),
        ],
        scratch_types=[
            pltpu.VMEM((S,), jnp.float32),
            pltpu.VMEM((K, D), jnp.float32),
            pltpu.VMEM((RPW * K + 16,), jnp.float32),
            pltpu.VMEM((RPW, K), jnp.int32),
            pltpu.VMEM((K,), jnp.int32),
            pltpu.VMEM((RPW, D), jnp.float32),
            pltpu.VMEM((D,), jnp.float32),
            pltpu.VMEM((D,), jnp.float32),
            pltpu.VMEM((16,), jnp.float32),
            pltpu.SemaphoreType.DMA,
            pltpu.SemaphoreType.DMA,
        ],
    )
    return f(vals.reshape(B * K), idx, wd2, x2, bd)


# speedup vs baseline: 9.0478x; 1.0007x over previous
import functools

import jax
import jax.numpy as jnp
from jax import lax
from jax.experimental import pallas as pl
from jax.experimental.pallas import tpu as pltpu
from jax.experimental.pallas import tpu_sc as plsc

B = 1024
T = 2
D_IN = 128
D = T * D_IN
S = 65536
K = 32
SEG = 512
NSEG = S // SEG
LEV = 10
NCAND = NSEG * LEV
NEG = float("-inf")

NW = 32
RPW = B // NW


def _encode_kernel(xmt_ref, w_ref, be_ref, mv_ref, mi_ref, work_ref):
    cb = pl.program_id(0)
    pre = lax.dot_general(
        w_ref[...], xmt_ref[...], (((1,), (0,)), ((), ())),
        preferred_element_type=jnp.float32,
        precision=lax.Precision.DEFAULT,
    )
    work_ref[...] = pre + be_ref[...].reshape(SEG, 1)
    base = cb * SEG

    def level(t, _):
        w = work_ref[...]
        m = jnp.max(w, axis=0)
        i = jnp.argmax(w, axis=0).astype(jnp.int32)
        mv_ref[0, t, 0, :] = m
        mi_ref[0, t, 0, :] = i + base
        iota0 = lax.broadcasted_iota(jnp.int32, (SEG, B), 0)
        work_ref[...] = jnp.where(iota0 == i[None, :], NEG, w)
        return 0

    lax.fori_loop(0, LEV, level, 0)


def _topk_kernel(mv_ref, mi_ref, ov_ref, oi_ref, cv_ref, sv_ref, si_ref):
    cv_ref[...] = mv_ref[...].reshape(NCAND, B)

    def pop(p, _):
        c = cv_ref[...]
        m = jnp.max(c, axis=0)
        fiota = lax.broadcasted_iota(jnp.int32, (NCAND, B), 0)
        eq = c == m[None, :]
        fi = jnp.min(jnp.where(eq, fiota, NCAND), axis=0)
        oh = fiota == fi[None, :]
        ci = mi_ref[...].reshape(NCAND, B)
        gi = jnp.sum(jnp.where(oh, ci, 0), axis=0)
        sv_ref[p, :] = m
        si_ref[p, :] = gi
        cv_ref[...] = jnp.where(oh, NEG, c)
        return 0

    lax.fori_loop(0, K, pop, 0)
    ov_ref[...] = sv_ref[...].T
    oi_ref[...] = si_ref[...].T


def _sc_zdec_kernel(vals_hbm, idx_hbm, wd_hbm, x_hbm, bd_hbm,
                    z_hbm, xh_hbm, lp_hbm,
                    zbuf, gbuf, vbuf, ibuf, irow, xbuf, bdbuf, acc, lacc,
                    gsem, zsem):
    wid = lax.axis_index("s") * 2 + lax.axis_index("c")
    r0 = wid * RPW
    pltpu.sync_copy(vals_hbm.at[pl.ds(r0 * K, RPW * K)], vbuf.at[pl.ds(0, RPW * K)])
    pltpu.sync_copy(idx_hbm.at[pl.ds(r0, RPW)], ibuf)
    pltpu.sync_copy(x_hbm.at[pl.ds(r0, RPW)], xbuf)
    pltpu.sync_copy(bd_hbm, bdbuf)
    lacc[...] = jnp.zeros((16,), jnp.float32)

    def zero16(i, _):
        zbuf[pl.ds(i * 16, 16)] = jnp.zeros((16,), jnp.float32)
        return 0

    lax.fori_loop(0, S // 16, zero16, 0)

    def row(r, _):
        irow[pl.ds(0, 16)] = ibuf[r, pl.ds(0, 16)]
        irow[pl.ds(16, 16)] = ibuf[r, pl.ds(16, 16)]
        gcopy = pltpu.async_copy(wd_hbm.at[irow], gbuf, gsem)
        iv0 = irow[pl.ds(0, 16)]
        iv1 = irow[pl.ds(16, 16)]
        rv0 = jnp.maximum(vbuf[pl.ds(r * K, 16)], 0.0)
        rv1 = jnp.maximum(vbuf[pl.ds(r * K + 16, 16)], 0.0)
        plsc.store_scatter(zbuf, [iv0], rv0)
        plsc.store_scatter(zbuf, [iv1], rv1)
        zcopy = pltpu.async_copy(zbuf, z_hbm.at[r0 + r], zsem)

        gcopy.wait()
        for c in range(D // 16):
            acc[pl.ds(c * 16, 16)] = bdbuf[pl.ds(c * 16, 16)]

        def dec(k, _):
            rvk = jnp.maximum(vbuf[pl.ds(r * K + k, 16)][0], 0.0)
            sp = jnp.full((16,), rvk, jnp.float32)
            for c in range(D // 16):
                sl = pl.ds(c * 16, 16)
                acc[sl] = acc[sl] + sp * gbuf[k, sl]
            return 0

        lax.fori_loop(0, K, dec, 0)
        for c in range(D // 16):
            sl = pl.ds(c * 16, 16)
            d = acc[sl] - xbuf[r, sl]
            lacc[...] = lacc[...] + d * d
        pltpu.sync_copy(acc, xh_hbm.at[r0 + r])

        zcopy.wait()
        z16 = jnp.zeros((16,), jnp.float32)
        plsc.store_scatter(zbuf, [iv0], z16)
        plsc.store_scatter(zbuf, [iv1], z16)
        return 0

    lax.fori_loop(0, RPW, row, 0)
    pltpu.sync_copy(lacc, lp_hbm.at[wid])


def _loss_kernel(lp_ref, loss_ref):
    loss_ref[...] = (jnp.sum(lp_ref[...]) * (1.0 / (B * T))).reshape(1, 1)


@jax.jit
def kernel(x, W_enc, b_enc, W_dec, b_dec):
    xm = (x - b_dec[None]).reshape(B, D)
    xmt = xm.T
    w2 = W_enc.reshape(S, D)
    be3 = b_enc.reshape(NSEG, SEG, 1)

    mv, mi = pl.pallas_call(
        _encode_kernel,
        grid=(NSEG,),
        in_specs=[
            pl.BlockSpec((D, B), lambda cb: (0, 0)),
            pl.BlockSpec((SEG, D), lambda cb: (cb, 0)),
            pl.BlockSpec((1, SEG, 1), lambda cb: (cb, 0, 0)),
        ],
        out_specs=[
            pl.BlockSpec((1, LEV, 1, B), lambda cb: (cb, 0, 0, 0)),
            pl.BlockSpec((1, LEV, 1, B), lambda cb: (cb, 0, 0, 0)),
        ],
        out_shape=[
            jax.ShapeDtypeStruct((NSEG, LEV, 1, B), jnp.float32),
            jax.ShapeDtypeStruct((NSEG, LEV, 1, B), jnp.int32),
        ],
        scratch_shapes=[pltpu.VMEM((SEG, B), jnp.float32)],
    )(xmt, w2, be3)

    vals, idx = pl.pallas_call(
        _topk_kernel,
        out_shape=[
            jax.ShapeDtypeStruct((B, K), jnp.float32),
            jax.ShapeDtypeStruct((B, K), jnp.int32),
        ],
        scratch_shapes=[
            pltpu.VMEM((NCAND, B), jnp.float32),
            pltpu.VMEM((K, B), jnp.float32),
            pltpu.VMEM((K, B), jnp.int32),
        ],
    )(mv, mi)

    z, xh, lp = _sc_zdec(vals, idx, W_dec.reshape(S, D), x.reshape(B, D),
                         b_dec.reshape(D))

    loss = pl.pallas_call(
        _loss_kernel,
        out_shape=jax.ShapeDtypeStruct((1, 1), jnp.float32),
    )(lp)

    return (loss.reshape(()), xh.reshape(B, T, D_IN), z)


def _sc_zdec(vals, idx, wd2, x2, bd):
    mesh = plsc.VectorSubcoreMesh(core_axis_name="c", subcore_axis_name="s")
    f = pl.kernel(
        _sc_zdec_kernel, mesh=mesh,
        compiler_params=pltpu.CompilerParams(needs_layout_passes=False),
        out_type=[
            jax.ShapeDtypeStruct((B, S), jnp.float32),
            jax.ShapeDtypeStruct((B, D), jnp.float32),
            jax.ShapeDtypeStruct((NW, 16), jnp.float32),
        ],
        scratch_types=[
            pltpu.VMEM((S,), jnp.float32),
            pltpu.VMEM((K, D), jnp.float32),
            pltpu.VMEM((RPW * K + 16,), jnp.float32),
            pltpu.VMEM((RPW, K), jnp.int32),
            pltpu.VMEM((K,), jnp.int32),
            pltpu.VMEM((RPW, D), jnp.float32),
            pltpu.VMEM((D,), jnp.float32),
            pltpu.VMEM((D,), jnp.float32),
            pltpu.VMEM((16,), jnp.float32),
            pltpu.SemaphoreType.DMA,
            pltpu.SemaphoreType.DMA,
        ],
    )
    return f(vals.reshape(B * K), idx, wd2, x2, bd)


# speedup vs baseline: 9.4638x; 1.0460x over previous
import functools

import jax
import jax.numpy as jnp
from jax import lax
from jax.experimental import pallas as pl
from jax.experimental.pallas import tpu as pltpu
from jax.experimental.pallas import tpu_sc as plsc

B = 1024
T = 2
D_IN = 128
D = T * D_IN
S = 65536
K = 32
SEG = 512
NSEG = S // SEG
LEV = 10
NCAND = NSEG * LEV
NEG = float("-inf")

NW = 32
RPW = B // NW


def _encode_topk_kernel(xmt_ref, w_ref, be_ref, ov_ref, oi_ref,
                        work_ref, smv_ref, smi_ref, sv_ref, si_ref):
    cb = pl.program_id(0)
    pre = lax.dot_general(
        w_ref[...], xmt_ref[...], (((1,), (0,)), ((), ())),
        preferred_element_type=jnp.float32,
        precision=lax.Precision.DEFAULT,
    )
    work_ref[...] = pre + be_ref[...].reshape(SEG, 1)
    base = cb * SEG

    def level(t, _):
        w = work_ref[...]
        m = jnp.max(w, axis=0)
        i = jnp.argmax(w, axis=0).astype(jnp.int32)
        smv_ref[cb * LEV + t, :] = m
        smi_ref[cb * LEV + t, :] = i + base
        iota0 = lax.broadcasted_iota(jnp.int32, (SEG, B), 0)
        work_ref[...] = jnp.where(iota0 == i[None, :], NEG, w)
        return 0

    lax.fori_loop(0, LEV, level, 0)

    @pl.when(cb == NSEG - 1)
    def _finish():
        def pop(p, _):
            c = smv_ref[...]
            m = jnp.max(c, axis=0)
            fiota = lax.broadcasted_iota(jnp.int32, (NCAND, B), 0)
            eq = c == m[None, :]
            fi = jnp.min(jnp.where(eq, fiota, NCAND), axis=0)
            oh = fiota == fi[None, :]
            gi = jnp.sum(jnp.where(oh, smi_ref[...], 0), axis=0)
            sv_ref[p, :] = m
            si_ref[p, :] = gi
            smv_ref[...] = jnp.where(oh, NEG, c)
            return 0

        lax.fori_loop(0, K, pop, 0)
        ov_ref[...] = sv_ref[...].T
        oi_ref[...] = si_ref[...].T


def _sc_zdec_kernel(vals_hbm, idx_hbm, wd_hbm, x_hbm, bd_hbm,
                    z_hbm, xh_hbm, lp_hbm,
                    zbuf, gbuf, vbuf, ibuf, irow, xbuf, bdbuf, acc, lacc,
                    gsem, zsem):
    wid = lax.axis_index("s") * 2 + lax.axis_index("c")
    r0 = wid * RPW
    pltpu.sync_copy(vals_hbm.at[pl.ds(r0 * K, RPW * K)], vbuf.at[pl.ds(0, RPW * K)])
    pltpu.sync_copy(idx_hbm.at[pl.ds(r0, RPW)], ibuf)
    pltpu.sync_copy(x_hbm.at[pl.ds(r0, RPW)], xbuf)
    pltpu.sync_copy(bd_hbm, bdbuf)
    lacc[...] = jnp.zeros((16,), jnp.float32)

    def zero16(i, _):
        zbuf[pl.ds(i * 16, 16)] = jnp.zeros((16,), jnp.float32)
        return 0

    lax.fori_loop(0, S // 16, zero16, 0)

    def row(r, _):
        irow[pl.ds(0, 16)] = ibuf[r, pl.ds(0, 16)]
        irow[pl.ds(16, 16)] = ibuf[r, pl.ds(16, 16)]
        gcopy = pltpu.async_copy(wd_hbm.at[irow], gbuf, gsem)
        iv0 = irow[pl.ds(0, 16)]
        iv1 = irow[pl.ds(16, 16)]
        rv0 = jnp.maximum(vbuf[pl.ds(r * K, 16)], 0.0)
        rv1 = jnp.maximum(vbuf[pl.ds(r * K + 16, 16)], 0.0)
        plsc.store_scatter(zbuf, [iv0], rv0)
        plsc.store_scatter(zbuf, [iv1], rv1)
        zcopy = pltpu.async_copy(zbuf, z_hbm.at[r0 + r], zsem)

        gcopy.wait()
        for c in range(D // 16):
            acc[pl.ds(c * 16, 16)] = bdbuf[pl.ds(c * 16, 16)]

        def dec(k, _):
            rvk = jnp.maximum(vbuf[pl.ds(r * K + k, 16)][0], 0.0)
            sp = jnp.full((16,), rvk, jnp.float32)
            for c in range(D // 16):
                sl = pl.ds(c * 16, 16)
                acc[sl] = acc[sl] + sp * gbuf[k, sl]
            return 0

        lax.fori_loop(0, K, dec, 0)
        for c in range(D // 16):
            sl = pl.ds(c * 16, 16)
            d = acc[sl] - xbuf[r, sl]
            lacc[...] = lacc[...] + d * d
        pltpu.sync_copy(acc, xh_hbm.at[r0 + r])

        zcopy.wait()
        z16 = jnp.zeros((16,), jnp.float32)
        plsc.store_scatter(zbuf, [iv0], z16)
        plsc.store_scatter(zbuf, [iv1], z16)
        return 0

    lax.fori_loop(0, RPW, row, 0)
    pltpu.sync_copy(lacc, lp_hbm.at[wid])


def _loss_kernel(lp_ref, loss_ref):
    loss_ref[...] = (jnp.sum(lp_ref[...]) * (1.0 / (B * T))).reshape(1, 1)


@jax.jit
def kernel(x, W_enc, b_enc, W_dec, b_dec):
    xm = (x - b_dec[None]).reshape(B, D)
    xmt = xm.T
    w2 = W_enc.reshape(S, D)
    be3 = b_enc.reshape(NSEG, SEG, 1)

    vals, idx = pl.pallas_call(
        _encode_topk_kernel,
        grid=(NSEG,),
        in_specs=[
            pl.BlockSpec((D, B), lambda cb: (0, 0)),
            pl.BlockSpec((SEG, D), lambda cb: (cb, 0)),
            pl.BlockSpec((1, SEG, 1), lambda cb: (cb, 0, 0)),
        ],
        out_specs=[
            pl.BlockSpec((B, K), lambda cb: (0, 0)),
            pl.BlockSpec((B, K), lambda cb: (0, 0)),
        ],
        out_shape=[
            jax.ShapeDtypeStruct((B, K), jnp.float32),
            jax.ShapeDtypeStruct((B, K), jnp.int32),
        ],
        scratch_shapes=[
            pltpu.VMEM((SEG, B), jnp.float32),
            pltpu.VMEM((NCAND, B), jnp.float32),
            pltpu.VMEM((NCAND, B), jnp.int32),
            pltpu.VMEM((K, B), jnp.float32),
            pltpu.VMEM((K, B), jnp.int32),
        ],
    )(xmt, w2, be3)

    z, xh, lp = _sc_zdec(vals, idx, W_dec.reshape(S, D), x.reshape(B, D),
                         b_dec.reshape(D))

    loss = pl.pallas_call(
        _loss_kernel,
        out_shape=jax.ShapeDtypeStruct((1, 1), jnp.float32),
    )(lp)

    return (loss.reshape(()), xh.reshape(B, T, D_IN), z)


def _sc_zdec(vals, idx, wd2, x2, bd):
    mesh = plsc.VectorSubcoreMesh(core_axis_name="c", subcore_axis_name="s")
    f = pl.kernel(
        _sc_zdec_kernel, mesh=mesh,
        compiler_params=pltpu.CompilerParams(needs_layout_passes=False),
        out_type=[
            jax.ShapeDtypeStruct((B, S), jnp.float32),
            jax.ShapeDtypeStruct((B, D), jnp.float32),
            jax.ShapeDtypeStruct((NW, 16), jnp.float32),
        ],
        scratch_types=[
            pltpu.VMEM((S,), jnp.float32),
            pltpu.VMEM((K, D), jnp.float32),
            pltpu.VMEM((RPW * K + 16,), jnp.float32),
            pltpu.VMEM((RPW, K), jnp.int32),
            pltpu.VMEM((K,), jnp.int32),
            pltpu.VMEM((RPW, D), jnp.float32),
            pltpu.VMEM((D,), jnp.float32),
            pltpu.VMEM((D,), jnp.float32),
            pltpu.VMEM((16,), jnp.float32),
            pltpu.SemaphoreType.DMA,
            pltpu.SemaphoreType.DMA,
        ],
    )
    return f(vals.reshape(B * K), idx, wd2, x2, bd)
